# baseline (speedup 1.0000x reference)
import jax
import jax.numpy as jnp
from jax.experimental import pallas as pl
from jax.experimental.pallas import tpu as pltpu

B, L, H = 1024, 50, 768
EPS = 0.1
N_B = 64


def _body(cls_ref, sid_ref, sta_ref, box_ref, sco_ref, w18_ref, gext_ref,
          gamma_ref, beta_ref, out_ref):
    c = cls_ref[...][:, :, None]
    s = sid_ref[...][:, :, None]
    t = sta_ref[...][:, :, None]
    cols = jax.lax.broadcasted_iota(jnp.int32, (N_B, L, 12), 2)
    oh = (((cols == c) & (cols >= 1))
          | ((cols == s + 3) & (cols >= 4))
          | ((cols == t + 6) & (cols >= 7))).astype(jnp.float32)
    box = box_ref[...]
    sco = sco_ref[...][:, :, None]
    w18 = w18_ref[...]
    gext = gext_ref[...]
    gamma = gamma_ref[...]
    beta = beta_ref[...]
    for b in range(N_B):
        o = oh[b]
        x = box[b]
        sc = sco[b]
        emb = (jnp.dot(o, w18[:12], preferred_element_type=jnp.float32)
               + jnp.dot(x, w18[12:16], preferred_element_type=jnp.float32)
               + sc * w18[16:17] + w18[17:18])
        q = (jnp.dot(o, gext[:12], preferred_element_type=jnp.float32)
             + jnp.dot(x, gext[12:16], preferred_element_type=jnp.float32)
             + sc * gext[16:17] + gext[17:18])
        mu = q[:, 18:19]
        s2 = (jnp.sum(q[:, :12] * o, axis=-1, keepdims=True)
              + jnp.sum(q[:, 12:16] * x, axis=-1, keepdims=True)
              + q[:, 16:17] * sc + q[:, 17:18])
        var = s2 - mu * mu
        out_ref[b] = (emb - mu) * jax.lax.rsqrt(var + EPS) * gamma + beta


def kernel(class_labels, bboxes, scores, sides, states, cat_table, side_table,
           state_table, W_box, b_box, W_score, b_score, gamma, beta):
    w18 = jnp.concatenate(
        [cat_table, side_table, state_table, W_box.T, W_score.T,
         (b_box + b_score).reshape(1, H)], axis=0)
    gext = jnp.concatenate(
        [w18 @ w18.T / H, jnp.mean(w18, axis=1, keepdims=True)],
        axis=1)
    gamma2 = gamma.reshape(1, H)
    beta2 = beta.reshape(1, H)

    grid = (B // N_B,)

    def tok2(i):
        return (i, 0)

    def tok3(i):
        return (i, 0, 0)

    def rep2(i):
        return (0, 0)

    return pl.pallas_call(
        _body,
        grid=grid,
        in_specs=[
            pl.BlockSpec((N_B, L), tok2),
            pl.BlockSpec((N_B, L), tok2),
            pl.BlockSpec((N_B, L), tok2),
            pl.BlockSpec((N_B, L, 4), tok3),
            pl.BlockSpec((N_B, L), tok2),
            pl.BlockSpec((18, H), rep2),
            pl.BlockSpec((18, 19), rep2),
            pl.BlockSpec((1, H), rep2),
            pl.BlockSpec((1, H), rep2),
        ],
        out_specs=pl.BlockSpec((N_B, L, H), tok3),
        out_shape=jax.ShapeDtypeStruct((B, L, H), jnp.float32),
        compiler_params=pltpu.CompilerParams(
            dimension_semantics=("parallel",)),
    )(class_labels, sides, states, bboxes, scores, w18, gext, gamma2, beta2)


# speedup vs baseline: 1.7296x; 1.7296x over previous
import jax
import jax.numpy as jnp
from jax.experimental import pallas as pl
from jax.experimental.pallas import tpu as pltpu

B, L, H = 1024, 50, 768
EPS = 0.1
N_B = 64


def _body(cls_ref, sid_ref, sta_ref, box_ref, sco_ref, w18_ref, gext_ref,
          gamma_ref, beta_ref, out_ref):
    c = cls_ref[...][:, :, None]
    s = sid_ref[...][:, :, None]
    t = sta_ref[...][:, :, None]
    cols = jax.lax.broadcasted_iota(jnp.int32, (N_B, L, 12), 2)
    oh = (((cols == c) & (cols >= 1))
          | ((cols == s + 3) & (cols >= 4))
          | ((cols == t + 6) & (cols >= 7)))
    feat = jnp.concatenate(
        [oh.astype(jnp.float32), box_ref[...], sco_ref[...][:, :, None],
         jnp.ones((N_B, L, 1), jnp.float32)], axis=-1)
    w18 = w18_ref[...]
    gext = gext_ref[...]
    gamma = gamma_ref[...]
    beta = beta_ref[...]
    for b in range(N_B):
        f = feat[b]
        emb = jnp.dot(f, w18, preferred_element_type=jnp.float32)
        q = jnp.dot(f, gext, preferred_element_type=jnp.float32)
        mu = q[:, 18:19]
        s2 = jnp.sum(q[:, :18] * f, axis=-1, keepdims=True)
        var = s2 - mu * mu
        out_ref[b] = (emb - mu) * jax.lax.rsqrt(var + EPS) * gamma + beta


def kernel(class_labels, bboxes, scores, sides, states, cat_table, side_table,
           state_table, W_box, b_box, W_score, b_score, gamma, beta):
    w18 = jnp.concatenate(
        [cat_table, side_table, state_table, W_box.T, W_score.T,
         (b_box + b_score).reshape(1, H)], axis=0)
    gext = jnp.concatenate(
        [w18 @ w18.T / H, jnp.mean(w18, axis=1, keepdims=True)],
        axis=1)
    gamma2 = gamma.reshape(1, H)
    beta2 = beta.reshape(1, H)

    grid = (B // N_B,)

    def tok2(i):
        return (i, 0)

    def tok3(i):
        return (i, 0, 0)

    def rep2(i):
        return (0, 0)

    return pl.pallas_call(
        _body,
        grid=grid,
        in_specs=[
            pl.BlockSpec((N_B, L), tok2),
            pl.BlockSpec((N_B, L), tok2),
            pl.BlockSpec((N_B, L), tok2),
            pl.BlockSpec((N_B, L, 4), tok3),
            pl.BlockSpec((N_B, L), tok2),
            pl.BlockSpec((18, H), rep2),
            pl.BlockSpec((18, 19), rep2),
            pl.BlockSpec((1, H), rep2),
            pl.BlockSpec((1, H), rep2),
        ],
        out_specs=pl.BlockSpec((N_B, L, H), tok3),
        out_shape=jax.ShapeDtypeStruct((B, L, H), jnp.float32),
        compiler_params=pltpu.CompilerParams(
            dimension_semantics=("parallel",)),
    )(class_labels, sides, states, bboxes, scores, w18, gext, gamma2, beta2)
